# baseline (speedup 1.0000x reference)
import math
import functools

import jax
import jax.numpy as jnp
from jax import lax
from jax.experimental import pallas as pl
from jax.experimental.pallas import tpu as pltpu
from jax.experimental.pallas import tpu_sc as plsc

NC = 2
NS = 16
NW = NC * NS
CH = 128
EMB = 64
WROW = 128

_mesh = plsc.VectorSubcoreMesh(core_axis_name="c", subcore_axis_name="s")


def _wid():
    return lax.axis_index("s") * NC + lax.axis_index("c")


def _tc_in_body(x_ref, w_ref, b_ref, o_ref):
    h = jnp.dot(x_ref[...], w_ref[...], preferred_element_type=jnp.float32)
    h = h + b_ref[...]
    o_ref[...] = jnp.where(h >= 0, h, 0.01 * h)


def _tc_in(x, w, b):
    return pl.pallas_call(
        _tc_in_body,
        out_shape=jax.ShapeDtypeStruct((x.shape[0], w.shape[1]), jnp.float32),
    )(x, w, b)


def _tc_b_body(n, h_ref, w_ref, deg_ref, y_ref, dinv_ref):
    xl = jnp.dot(h_ref[...], w_ref[...], preferred_element_type=jnp.float32)
    deg = deg_ref[0, :n, 0:1] + deg_ref[1, :n, 0:1] + 1.0
    dinv = lax.rsqrt(deg)
    dinv_ref[...] = dinv
    y_ref[:, :EMB] = dinv * xl
    y_ref[:, EMB:] = jnp.zeros((n, WROW - EMB), jnp.float32)


def _tc_b(h, w, deg2):
    n = h.shape[0]
    return pl.pallas_call(
        functools.partial(_tc_b_body, n),
        out_shape=(jax.ShapeDtypeStruct((n, WROW), jnp.float32),
                   jax.ShapeDtypeStruct((n, 1), jnp.float32)),
    )(h, w, deg2)


def _tc_c1_body(n, np_, msg_ref, y_ref, dinv_ref, b_ref, g_ref, bb_ref, w_ref,
                hs_ref, sc_ref):
    y = y_ref[:, :EMB]
    agg = dinv_ref[...] * (msg_ref[0, :n, :EMB] + msg_ref[1, :n, :EMB] + y) + b_ref[...]
    m = jnp.mean(agg, axis=-1, keepdims=True)
    c = agg - m
    v = jnp.mean(c * c, axis=-1, keepdims=True)
    h = c / jnp.sqrt(v + 1e-5) * g_ref[...] + bb_ref[...]
    h = jnp.where(h >= 0, h, 0.01 * h)
    w = w_ref[...]
    wn = jnp.sqrt(jnp.sum(w * w))
    score = jnp.dot(h, w, preferred_element_type=jnp.float32) / wn
    hs_ref[:n, :EMB] = h * jnp.tanh(score)
    hs_ref[:n, EMB:] = jnp.zeros((n, WROW - EMB), jnp.float32)
    hs_ref[n:np_, :] = jnp.zeros((np_ - n, WROW), jnp.float32)
    sc_ref[:n, :] = score
    sc_ref[n:np_, :] = jnp.full((np_ - n, 1), -jnp.inf, jnp.float32)


def _tc_c1(msg2, y, dinv, b, g, bb, w, np_):
    n = dinv.shape[0]
    return pl.pallas_call(
        functools.partial(_tc_c1_body, n, np_),
        out_shape=(jax.ShapeDtypeStruct((np_, WROW), jnp.float32),
                   jax.ShapeDtypeStruct((np_, 1), jnp.float32)),
    )(msg2, y, dinv, b, g, bb, w)


_RB = 1024


def _tc_rank_body(np_, scol_ref, srow_ref, rank_ref):
    nb = np_ // _RB

    def body(t, _):
        ib = t // nb
        jb = t % nb
        si = scol_ref[pl.ds(ib * _RB, _RB), :]
        sj = srow_ref[:, pl.ds(jb * _RB, _RB)]
        ii = ib * _RB + lax.broadcasted_iota(jnp.int32, (_RB, _RB), 0)
        jj = jb * _RB + lax.broadcasted_iota(jnp.int32, (_RB, _RB), 1)
        cmp = (sj > si) | ((sj == si) & (jj < ii))
        p = jnp.sum(cmp.astype(jnp.int32), axis=1, keepdims=True)
        prev = rank_ref[pl.ds(ib * _RB, _RB), :]
        rank_ref[pl.ds(ib * _RB, _RB), :] = jnp.where(jb == 0, p, prev + p)
        return 0

    lax.fori_loop(0, nb * nb, body, 0)


def _tc_rank(scol, srow):
    np_ = scol.shape[0]
    return pl.pallas_call(
        functools.partial(_tc_rank_body, np_),
        out_shape=jax.ShapeDtypeStruct((np_, 1), jnp.int32),
    )(scol, srow)


def _sc_deg(dst2, ones16, zeros16, n_acc, rows_per_tile):
    stripe = n_acc // NS

    @functools.partial(
        pl.kernel, mesh=_mesh,
        out_type=jax.ShapeDtypeStruct((NC, n_acc, 16), jnp.float32),
        scratch_types=[
            pltpu.VMEM((rows_per_tile, CH), jnp.int32),
            pltpu.VMEM((CH, 16), jnp.float32),
            pltpu.VMEM_SHARED((n_acc, 16), jnp.float32),
            pltpu.SemaphoreType.DMA,
        ],
    )
    def k(dst_hbm, ones_hbm, zeros_hbm, out_hbm, idx_v, ones_v, acc, dsem):
        cid = lax.axis_index("c")
        sid = lax.axis_index("s")
        wid = sid * NC + cid
        off = sid * stripe
        pltpu.sync_copy(zeros_hbm.at[pl.ds(off, stripe)], acc.at[pl.ds(off, stripe)])
        pltpu.sync_copy(ones_hbm, ones_v)
        pltpu.sync_copy(dst_hbm.at[pl.ds(wid * rows_per_tile, rows_per_tile)], idx_v)
        plsc.subcore_barrier()

        def body(j, _):
            pltpu.async_copy(ones_v, acc.at[idx_v.at[j]], dsem, add=True)
            return 0

        lax.fori_loop(0, rows_per_tile, body, 0)

        def drain(j, _):
            pltpu.make_async_copy(ones_v, acc.at[idx_v.at[0]], dsem).wait()
            return 0

        lax.fori_loop(0, rows_per_tile, drain, 0)
        plsc.subcore_barrier()
        pltpu.sync_copy(acc.at[pl.ds(off, stripe)],
                        out_hbm.at[cid, pl.ds(off, stripe)])

    return k(dst2, ones16, zeros16)


def _sc_msg(src2, dst2, y, zeros, half, lo, rows_per_tile):
    stripe = half // NS
    nzfull, zrem = stripe // CH, stripe % CH

    @functools.partial(
        pl.kernel, mesh=_mesh,
        out_type=jax.ShapeDtypeStruct((NC, half, WROW), jnp.float32),
        scratch_types=[
            pltpu.VMEM((rows_per_tile, CH), jnp.int32),
            pltpu.VMEM((rows_per_tile, CH), jnp.int32),
            pltpu.VMEM((4, CH, WROW), jnp.float32),
            pltpu.VMEM_SHARED((half + 8, WROW), jnp.float32),
        ] + [pltpu.SemaphoreType.DMA] * 8,
    )
    def k(src_hbm, dst_hbm, y_hbm, zeros_hbm, out_hbm,
          src_v, dst_v, rows_v, acc, g0, g1, g2, g3, s0, s1, s2, s3):
        gsem = [g0, g1, g2, g3]
        ssem = [s0, s1, s2, s3]
        cid = lax.axis_index("c")
        sid = lax.axis_index("s")
        wid = sid * NC + cid
        off = sid * stripe
        for z in range(nzfull):
            pltpu.sync_copy(zeros_hbm, acc.at[pl.ds(off + z * CH, CH)])
        if zrem:
            pltpu.sync_copy(zeros_hbm.at[pl.ds(0, zrem)],
                            acc.at[pl.ds(off + nzfull * CH, zrem)])
        pltpu.sync_copy(src_hbm.at[pl.ds(wid * rows_per_tile, rows_per_tile)], src_v)
        pltpu.sync_copy(dst_hbm.at[pl.ds(wid * rows_per_tile, rows_per_tile)], dst_v)

        def remap(r, _):
            for j in range(CH // 16):
                d = dst_v[r, pl.ds(j * 16, 16)]
                dl = d - lo
                ok = (dl >= 0) & (dl < half)
                dst_v[r, pl.ds(j * 16, 16)] = jnp.where(ok, dl, half)
            return 0

        lax.fori_loop(0, rows_per_tile, remap, 0)
        plsc.subcore_barrier()

        NB = 4
        for b in range(NB):
            pltpu.async_copy(y_hbm.at[src_v.at[b]], rows_v.at[b], gsem[b])

        def body(j4, _):
            for b in range(NB):
                j = j4 * NB + b
                pltpu.make_async_copy(y_hbm.at[src_v.at[j]], rows_v.at[b],
                                      gsem[b]).wait()
                pltpu.async_copy(rows_v.at[b], acc.at[dst_v.at[j]], ssem[b],
                                 add=True)

                @pl.when(j + NB < rows_per_tile)
                def _():
                    pltpu.make_async_copy(rows_v.at[b], acc.at[dst_v.at[j]],
                                          ssem[b]).wait()
                    pltpu.async_copy(y_hbm.at[src_v.at[j + NB]], rows_v.at[b],
                                     gsem[b])
            return 0

        lax.fori_loop(0, rows_per_tile // NB, body, 0)
        for b in range(NB):
            pltpu.make_async_copy(rows_v.at[b], acc.at[dst_v.at[0]],
                                  ssem[b]).wait()
        plsc.subcore_barrier()
        pltpu.sync_copy(acc.at[pl.ds(off, stripe)],
                        out_hbm.at[cid, pl.ds(off, stripe)])

    return k(src2, dst2, y, zeros)


def _sc_pool(hs, rank, srcp, dstp, n, k_new, np_, e_pad):
    ept = e_pad // NW
    node_rows = np_ // NW
    nchunks = node_rows // CH

    @functools.partial(
        pl.kernel, mesh=_mesh,
        compiler_params=pltpu.CompilerParams(needs_layout_passes=False),
        out_type=(jax.ShapeDtypeStruct((k_new + 1, WROW), jnp.float32),
                  jax.ShapeDtypeStruct((e_pad,), jnp.int32),
                  jax.ShapeDtypeStruct((e_pad,), jnp.int32)),
        scratch_types=[
            pltpu.VMEM((np_,), jnp.int32),
            pltpu.VMEM((nchunks, CH), jnp.int32),
            pltpu.VMEM((CH, WROW), jnp.float32),
            pltpu.VMEM((ept,), jnp.int32),
            pltpu.VMEM((ept,), jnp.int32),
            pltpu.VMEM((ept,), jnp.int32),
            pltpu.VMEM((ept,), jnp.int32),
        ],
    )
    def k(hs_hbm, rank_hbm, src_hbm, dst_hbm,
          hnew_hbm, srcn_hbm, dstn_hbm,
          rank_v, ridx_v, rows_v, src_v, dst_v, srco_v, dsto_v):
        cid = lax.axis_index("c")
        sid = lax.axis_index("s")
        wid = sid * NC + cid
        pltpu.sync_copy(rank_hbm, rank_v)

        nbase = wid * node_rows
        for c in range(nchunks):
            for j in range(CH // 16):
                r = rank_v[pl.ds(nbase + c * CH + j * 16, 16)]
                ridx_v[c, pl.ds(j * 16, 16)] = jnp.minimum(r, k_new)
            pltpu.sync_copy(hs_hbm.at[pl.ds(nbase + c * CH, CH)], rows_v)
            pltpu.sync_copy(rows_v, hnew_hbm.at[ridx_v.at[c]])

        ebase = wid * ept
        pltpu.sync_copy(src_hbm.at[pl.ds(ebase, ept)], src_v)
        pltpu.sync_copy(dst_hbm.at[pl.ds(ebase, ept)], dst_v)

        def body(j, _):
            s = src_v[pl.ds(j * 16, 16)]
            d = dst_v[pl.ds(j * 16, 16)]
            rs = plsc.load_gather(rank_v, [s])
            rd = plsc.load_gather(rank_v, [d])
            keep = (rs < k_new) & (rd < k_new)
            srco_v[pl.ds(j * 16, 16)] = jnp.where(keep, rs, 0)
            dsto_v[pl.ds(j * 16, 16)] = jnp.where(keep, rd, k_new)
            return 0

        lax.fori_loop(0, ept // 16, body, 0)
        pltpu.sync_copy(srco_v, srcn_hbm.at[pl.ds(ebase, ept)])
        pltpu.sync_copy(dsto_v, dstn_hbm.at[pl.ds(ebase, ept)])

    return k(hs, rank, srcp, dstp)


def _sizes(n):
    n_acc = 128 * ((n + 1 + 127) // 128)
    np_ = NW * CH * max(1, -(-(n + 1) // (NW * CH)))
    return n_acc, np_


def kernel(x, edge_index, params):
    n = x.shape[0]
    e = edge_index.shape[1]
    e_pad = NW * CH * 8 * (-(-e // (NW * CH * 8)))
    src = edge_index[0].astype(jnp.int32)
    dst = edge_index[1].astype(jnp.int32)
    srcp = jnp.concatenate([src, jnp.zeros((e_pad - e,), jnp.int32)])
    dstp = jnp.concatenate([dst, jnp.full((e_pad - e,), n, jnp.int32)])
    ones16 = jnp.ones((CH, 16), jnp.float32)
    rpt = e_pad // NW // CH

    h = _tc_in(x, params['W_in'], params['b_in'].reshape(1, EMB))

    n_layers = params['W'].shape[0]
    for i in range(n_layers):
        k = int(math.ceil(0.5 * n))
        n_acc, np_ = _sizes(n)
        src2 = srcp.reshape(e_pad // CH, CH)
        dst2 = dstp.reshape(e_pad // CH, CH)

        deg2 = _sc_deg(dst2, ones16, jnp.zeros((n_acc, 16), jnp.float32),
                       n_acc, rpt)
        y, dinv = _tc_b(h, params['W'][i], deg2)
        zt = jnp.zeros((CH, WROW), jnp.float32)
        if n_acc <= 8192:
            msg2 = _sc_msg(src2, dst2, y, zt, n_acc, 0, rpt)
        else:
            half = 128 * (-(-(n + 1) // 256))
            msg2 = jnp.concatenate(
                [_sc_msg(src2, dst2, y, zt, half, 0, rpt),
                 _sc_msg(src2, dst2, y, zt, half, half, rpt)], axis=1)
            n_acc = 2 * half
        hs, scol = _tc_c1(msg2, y, dinv,
                          params['b'][i].reshape(1, EMB),
                          params['ln_g'][i].reshape(1, EMB),
                          params['ln_b'][i].reshape(1, EMB),
                          params['pool_w'][i].reshape(EMB, 1), np_)
        rank = _tc_rank(scol, scol.reshape(1, np_)).reshape(np_)
        hnew, srcp, dstp = _sc_pool(hs, rank, srcp, dstp, n, k, np_, e_pad)
        h = hnew[:k, :EMB]
        n = k
    return h


# speedup vs baseline: 29.1595x; 29.1595x over previous
import math
import functools

import jax
import jax.numpy as jnp
from jax import lax
from jax.experimental import pallas as pl
from jax.experimental.pallas import tpu as pltpu
from jax.experimental.pallas import tpu_sc as plsc

NC = 2
NS = 16
NW = NC * NS
CH = 128
EMB = 64
WROW = 128

_mesh = plsc.VectorSubcoreMesh(core_axis_name="c", subcore_axis_name="s")


def _wid():
    return lax.axis_index("s") * NC + lax.axis_index("c")


def _tc_in_body(x_ref, w_ref, b_ref, o_ref):
    h = jnp.dot(x_ref[...], w_ref[...], preferred_element_type=jnp.float32)
    h = h + b_ref[...]
    o_ref[...] = jnp.where(h >= 0, h, 0.01 * h)


def _tc_in(x, w, b):
    return pl.pallas_call(
        _tc_in_body,
        out_shape=jax.ShapeDtypeStruct((x.shape[0], w.shape[1]), jnp.float32),
    )(x, w, b)


def _tc_b_body(n, h_ref, w_ref, deg_ref, y_ref, dinv_ref):
    xl = jnp.dot(h_ref[...], w_ref[...], preferred_element_type=jnp.float32)
    deg = deg_ref[0, :n, 0:1] + deg_ref[1, :n, 0:1] + 1.0
    dinv = lax.rsqrt(deg)
    dinv_ref[...] = dinv
    y_ref[:, :EMB] = dinv * xl
    y_ref[:, EMB:] = jnp.zeros((n, WROW - EMB), jnp.float32)


def _tc_b(h, w, deg2):
    n = h.shape[0]
    return pl.pallas_call(
        functools.partial(_tc_b_body, n),
        out_shape=(jax.ShapeDtypeStruct((n, WROW), jnp.float32),
                   jax.ShapeDtypeStruct((n, 1), jnp.float32)),
    )(h, w, deg2)


def _tc_c1_body(n, np_, msg_ref, y_ref, dinv_ref, b_ref, g_ref, bb_ref, w_ref,
                hs_ref, sc_ref):
    y = y_ref[:, :EMB]
    agg = dinv_ref[...] * (msg_ref[0, :n, :EMB] + msg_ref[1, :n, :EMB] + y) + b_ref[...]
    m = jnp.mean(agg, axis=-1, keepdims=True)
    c = agg - m
    v = jnp.mean(c * c, axis=-1, keepdims=True)
    h = c / jnp.sqrt(v + 1e-5) * g_ref[...] + bb_ref[...]
    h = jnp.where(h >= 0, h, 0.01 * h)
    w = w_ref[...]
    wn = jnp.sqrt(jnp.sum(w * w))
    score = jnp.dot(h, w, preferred_element_type=jnp.float32) / wn
    hs_ref[:n, :EMB] = h * jnp.tanh(score)
    hs_ref[:n, EMB:] = jnp.zeros((n, WROW - EMB), jnp.float32)
    hs_ref[n:np_, :] = jnp.zeros((np_ - n, WROW), jnp.float32)
    sc_ref[:n, :] = score
    sc_ref[n:np_, :] = jnp.full((np_ - n, 1), -jnp.inf, jnp.float32)


def _tc_c1(msg2, y, dinv, b, g, bb, w, np_):
    n = dinv.shape[0]
    return pl.pallas_call(
        functools.partial(_tc_c1_body, n, np_),
        out_shape=(jax.ShapeDtypeStruct((np_, WROW), jnp.float32),
                   jax.ShapeDtypeStruct((np_, 1), jnp.float32)),
    )(msg2, y, dinv, b, g, bb, w)


_RB = 1024


def _tc_rank_body(np_, scol_ref, srow_ref, rank_ref):
    nb = np_ // _RB

    def body(t, _):
        ib = t // nb
        jb = t % nb
        si = scol_ref[pl.ds(ib * _RB, _RB), :]
        sj = srow_ref[:, pl.ds(jb * _RB, _RB)]
        ii = ib * _RB + lax.broadcasted_iota(jnp.int32, (_RB, _RB), 0)
        jj = jb * _RB + lax.broadcasted_iota(jnp.int32, (_RB, _RB), 1)
        cmp = (sj > si) | ((sj == si) & (jj < ii))
        p = jnp.sum(cmp.astype(jnp.int32), axis=1, keepdims=True)
        prev = rank_ref[pl.ds(ib * _RB, _RB), :]
        rank_ref[pl.ds(ib * _RB, _RB), :] = jnp.where(jb == 0, p, prev + p)
        return 0

    lax.fori_loop(0, nb * nb, body, 0)


def _tc_rank(scol, srow):
    np_ = scol.shape[0]
    return pl.pallas_call(
        functools.partial(_tc_rank_body, np_),
        out_shape=jax.ShapeDtypeStruct((np_, 1), jnp.int32),
    )(scol, srow)


def _sc_deg(dst2, ones16, zeros16, n_acc, rows_per_tile):
    stripe = n_acc // NS

    @functools.partial(
        pl.kernel, mesh=_mesh,
        out_type=jax.ShapeDtypeStruct((NC, n_acc, 16), jnp.float32),
        scratch_types=[
            pltpu.VMEM((rows_per_tile, CH), jnp.int32),
            pltpu.VMEM((CH, 16), jnp.float32),
            pltpu.VMEM_SHARED((n_acc, 16), jnp.float32),
            pltpu.SemaphoreType.DMA,
        ],
    )
    def k(dst_hbm, ones_hbm, zeros_hbm, out_hbm, idx_v, ones_v, acc, dsem):
        cid = lax.axis_index("c")
        sid = lax.axis_index("s")
        wid = sid * NC + cid
        off = sid * stripe
        pltpu.sync_copy(zeros_hbm.at[pl.ds(off, stripe)], acc.at[pl.ds(off, stripe)])
        pltpu.sync_copy(ones_hbm, ones_v)
        pltpu.sync_copy(dst_hbm.at[pl.ds(wid * rows_per_tile, rows_per_tile)], idx_v)
        plsc.subcore_barrier()

        def body(j, _):
            pltpu.async_copy(ones_v, acc.at[idx_v.at[j]], dsem, add=True)
            return 0

        lax.fori_loop(0, rows_per_tile, body, 0)

        def drain(j, _):
            pltpu.make_async_copy(ones_v, acc.at[idx_v.at[0]], dsem).wait()
            return 0

        lax.fori_loop(0, rows_per_tile, drain, 0)
        plsc.subcore_barrier()
        pltpu.sync_copy(acc.at[pl.ds(off, stripe)],
                        out_hbm.at[cid, pl.ds(off, stripe)])

    return k(dst2, ones16, zeros16)


def _sc_msg(src2, dst2, y, zeros, half, lo, rows_per_tile):
    stripe = half // NS
    nzfull, zrem = stripe // CH, stripe % CH

    @functools.partial(
        pl.kernel, mesh=_mesh,
        out_type=jax.ShapeDtypeStruct((NC, half, WROW), jnp.float32),
        scratch_types=[
            pltpu.VMEM((rows_per_tile, CH), jnp.int32),
            pltpu.VMEM((rows_per_tile, CH), jnp.int32),
            pltpu.VMEM((4, CH, WROW), jnp.float32),
            pltpu.VMEM_SHARED((half + 128, WROW), jnp.float32),
        ] + [pltpu.SemaphoreType.DMA] * 8,
    )
    def k(src_hbm, dst_hbm, y_hbm, zeros_hbm, out_hbm,
          src_v, dst_v, rows_v, acc, g0, g1, g2, g3, s0, s1, s2, s3):
        gsem = [g0, g1, g2, g3]
        ssem = [s0, s1, s2, s3]
        cid = lax.axis_index("c")
        sid = lax.axis_index("s")
        wid = sid * NC + cid
        off = sid * stripe
        for z in range(nzfull):
            pltpu.sync_copy(zeros_hbm, acc.at[pl.ds(off + z * CH, CH)])
        if zrem:
            pltpu.sync_copy(zeros_hbm.at[pl.ds(0, zrem)],
                            acc.at[pl.ds(off + nzfull * CH, zrem)])
        pltpu.sync_copy(src_hbm.at[pl.ds(wid * rows_per_tile, rows_per_tile)], src_v)
        pltpu.sync_copy(dst_hbm.at[pl.ds(wid * rows_per_tile, rows_per_tile)], dst_v)

        lane = lax.iota(jnp.int32, 16)

        def remap(r, _):
            for j in range(CH // 16):
                d = dst_v[r, pl.ds(j * 16, 16)]
                dl = d - lo
                ok = (dl >= 0) & (dl < half)
                trash = half + ((r * CH + j * 16 + lane) & 127)
                dst_v[r, pl.ds(j * 16, 16)] = jnp.where(ok, dl, trash)
            return 0

        lax.fori_loop(0, rows_per_tile, remap, 0)
        plsc.subcore_barrier()

        NB = 4
        for b in range(NB):
            pltpu.async_copy(y_hbm.at[src_v.at[b]], rows_v.at[b], gsem[b])

        def body(j4, _):
            for b in range(NB):
                j = j4 * NB + b
                pltpu.make_async_copy(y_hbm.at[src_v.at[j]], rows_v.at[b],
                                      gsem[b]).wait()
                pltpu.async_copy(rows_v.at[b], acc.at[dst_v.at[j]], ssem[b],
                                 add=True)

                @pl.when(j + NB < rows_per_tile)
                def _():
                    pltpu.make_async_copy(rows_v.at[b], acc.at[dst_v.at[j]],
                                          ssem[b]).wait()
                    pltpu.async_copy(y_hbm.at[src_v.at[j + NB]], rows_v.at[b],
                                     gsem[b])
            return 0

        lax.fori_loop(0, rows_per_tile // NB, body, 0)
        for b in range(NB):
            pltpu.make_async_copy(rows_v.at[b], acc.at[dst_v.at[0]],
                                  ssem[b]).wait()
        plsc.subcore_barrier()
        pltpu.sync_copy(acc.at[pl.ds(off, stripe)],
                        out_hbm.at[cid, pl.ds(off, stripe)])

    return k(src2, dst2, y, zeros)


def _sc_pool(hs, rank, srcp, dstp, n, k_new, np_, e_pad):
    ept = e_pad // NW
    node_rows = np_ // NW
    nchunks = node_rows // CH

    tn = 128 * (-(-(k_new + 1) // 128))

    @functools.partial(
        pl.kernel, mesh=_mesh,
        compiler_params=pltpu.CompilerParams(needs_layout_passes=False),
        out_type=(jax.ShapeDtypeStruct((k_new + 128, WROW), jnp.float32),
                  jax.ShapeDtypeStruct((e_pad,), jnp.int32),
                  jax.ShapeDtypeStruct((e_pad,), jnp.int32)),
        scratch_types=[
            pltpu.VMEM((np_,), jnp.int32),
            pltpu.VMEM((nchunks, CH), jnp.int32),
            pltpu.VMEM((CH, WROW), jnp.float32),
            pltpu.VMEM((ept,), jnp.int32),
            pltpu.VMEM((ept,), jnp.int32),
            pltpu.VMEM((ept,), jnp.int32),
            pltpu.VMEM((ept,), jnp.int32),
        ],
    )
    def k(hs_hbm, rank_hbm, src_hbm, dst_hbm,
          hnew_hbm, srcn_hbm, dstn_hbm,
          rank_v, ridx_v, rows_v, src_v, dst_v, srco_v, dsto_v):
        cid = lax.axis_index("c")
        sid = lax.axis_index("s")
        wid = sid * NC + cid
        pltpu.sync_copy(rank_hbm, rank_v)

        nbase = wid * node_rows
        for c in range(nchunks):
            for j in range(CH // 16):
                r = rank_v[pl.ds(nbase + c * CH + j * 16, 16)]
                lane = lax.iota(jnp.int32, 16)
                ridx_v[c, pl.ds(j * 16, 16)] = jnp.where(
                    r < k_new, r, k_new + ((r + lane) & 127))
            pltpu.sync_copy(hs_hbm.at[pl.ds(nbase + c * CH, CH)], rows_v)
            pltpu.sync_copy(rows_v, hnew_hbm.at[ridx_v.at[c]])

        ebase = wid * ept
        pltpu.sync_copy(src_hbm.at[pl.ds(ebase, ept)], src_v)
        pltpu.sync_copy(dst_hbm.at[pl.ds(ebase, ept)], dst_v)

        lane = lax.iota(jnp.int32, 16)

        def body(j, _):
            s = src_v[pl.ds(j * 16, 16)]
            d = dst_v[pl.ds(j * 16, 16)]
            rs = plsc.load_gather(rank_v, [s])
            rd = plsc.load_gather(rank_v, [d])
            keep = (rs < k_new) & (rd < k_new)
            spread = j * 16 + lane
            srco_v[pl.ds(j * 16, 16)] = jnp.where(keep, rs, spread & 511)
            dsto_v[pl.ds(j * 16, 16)] = jnp.where(keep, rd, tn + (spread & 127))
            return 0

        lax.fori_loop(0, ept // 16, body, 0)
        pltpu.sync_copy(srco_v, srcn_hbm.at[pl.ds(ebase, ept)])
        pltpu.sync_copy(dsto_v, dstn_hbm.at[pl.ds(ebase, ept)])

    return k(hs, rank, srcp, dstp)


def _sizes(n):
    n_acc = 128 * ((n + 1 + 127) // 128)
    np_ = NW * CH * max(1, -(-(n + 1) // (NW * CH)))
    return n_acc, np_


def kernel(x, edge_index, params):
    n = x.shape[0]
    e = edge_index.shape[1]
    e_pad = NW * CH * 8 * (-(-e // (NW * CH * 8)))
    src = edge_index[0].astype(jnp.int32)
    dst = edge_index[1].astype(jnp.int32)
    t0 = 128 * (-(-(n + 1) // 128))
    pad_i = jnp.arange(e_pad - e, dtype=jnp.int32)
    srcp = jnp.concatenate([src, pad_i & 511])
    dstp = jnp.concatenate([dst, t0 + (pad_i & 127)])
    ones16 = jnp.ones((CH, 16), jnp.float32)
    rpt = e_pad // NW // CH

    h = _tc_in(x, params['W_in'], params['b_in'].reshape(1, EMB))

    n_layers = params['W'].shape[0]
    for i in range(n_layers):
        k = int(math.ceil(0.5 * n))
        n_acc, np_ = _sizes(n)
        src2 = srcp.reshape(e_pad // CH, CH)
        dst2 = dstp.reshape(e_pad // CH, CH)

        deg2 = _sc_deg(dst2, ones16, jnp.zeros((n_acc + 128, 16), jnp.float32),
                       n_acc + 128, rpt)
        y, dinv = _tc_b(h, params['W'][i], deg2)
        zt = jnp.zeros((CH, WROW), jnp.float32)
        if n_acc <= 8192:
            msg2 = _sc_msg(src2, dst2, y, zt, n_acc, 0, rpt)
        else:
            half = 128 * (-(-(n + 1) // 256))
            msg2 = jnp.concatenate(
                [_sc_msg(src2, dst2, y, zt, half, 0, rpt),
                 _sc_msg(src2, dst2, y, zt, half, half, rpt)], axis=1)
            n_acc = 2 * half
        hs, scol = _tc_c1(msg2, y, dinv,
                          params['b'][i].reshape(1, EMB),
                          params['ln_g'][i].reshape(1, EMB),
                          params['ln_b'][i].reshape(1, EMB),
                          params['pool_w'][i].reshape(EMB, 1), np_)
        rank = _tc_rank(scol, scol.reshape(1, np_)).reshape(np_)
        hnew, srcp, dstp = _sc_pool(hs, rank, srcp, dstp, n, k, np_, e_pad)
        h = hnew[:k, :EMB]
        n = k
    return h
